# baseline (speedup 1.0000x reference)
import functools

import jax
import jax.numpy as jnp
from jax.experimental import pallas as pl

N = 10000
E = 160000
DOUT = 256
G = 64
NCONV = 3
SIZE = 256

_BN_SCALE = 1.0 / (1.0 + 1e-5) ** 0.5


def _act(y, kind):
    if kind == "none":
        return y
    if kind == "lrelu":
        return jnp.where(y > 0, y, 0.01 * y)
    if kind == "lrelu2":
        return jnp.where(y > 0, y, 0.0001 * y)
    if kind == "sigmoid":
        return jax.nn.sigmoid(y)
    raise ValueError(kind)


def _mm_body(x_ref, w_ref, b_ref, o_ref, *, kind):
    y = jnp.dot(x_ref[...], w_ref[...], preferred_element_type=jnp.float32)
    o_ref[...] = _act(y + b_ref[...], kind)


def _mm(x, w, b, kind="none", bm=2000):
    m, k = x.shape
    n = w.shape[1]
    grid = (m + bm - 1) // bm
    return pl.pallas_call(
        functools.partial(_mm_body, kind=kind),
        grid=(grid,),
        in_specs=[
            pl.BlockSpec((bm, k), lambda i: (i, 0)),
            pl.BlockSpec((k, n), lambda i: (0, 0)),
            pl.BlockSpec((1, n), lambda i: (0, 0)),
        ],
        out_specs=pl.BlockSpec((bm, n), lambda i: (i, 0)),
        out_shape=jax.ShapeDtypeStruct((m, n), jnp.float32),
    )(x, w, b.reshape(1, n))


def _mlp2_body(x_ref, w1_ref, b1_ref, w2_ref, b2_ref, o_ref):
    y = jnp.dot(x_ref[...], w1_ref[...], preferred_element_type=jnp.float32)
    y = _act(y + b1_ref[...], "lrelu")
    z = jnp.dot(y, w2_ref[...], preferred_element_type=jnp.float32)
    o_ref[...] = _act(z + b2_ref[...], "lrelu2")


def _mlp2(x, w1, b1, w2, b2, bm=2000):
    m, k = x.shape
    n = w1.shape[1]
    grid = (m + bm - 1) // bm
    return pl.pallas_call(
        _mlp2_body,
        grid=(grid,),
        in_specs=[
            pl.BlockSpec((bm, k), lambda i: (i, 0)),
            pl.BlockSpec((k, n), lambda i: (0, 0)),
            pl.BlockSpec((1, n), lambda i: (0, 0)),
            pl.BlockSpec((n, n), lambda i: (0, 0)),
            pl.BlockSpec((1, n), lambda i: (0, 0)),
        ],
        out_specs=pl.BlockSpec((bm, n), lambda i: (i, 0)),
        out_shape=jax.ShapeDtypeStruct((m, n), jnp.float32),
    )(x, w1, b1.reshape(1, n), w2, b2.reshape(1, n))


def _fold_bn(w, b, g, be):
    s = g * _BN_SCALE
    return w * s[None, :], b * s + be


def kernel(x, edge_index, edge_attr, batch, params):
    p = params
    src = edge_index[0]
    dst = edge_index[1]

    hs = []
    h = x
    for i in range(NCONV + 1):
        eproj = _mm(edge_attr, p["c%d_eW" % i], p["c%d_eb" % i], bm=8000)
        m = jax.nn.relu(h[src] + eproj)
        aggr = jax.ops.segment_sum(m, dst, num_segments=N)
        w1, b1 = _fold_bn(p["c%d_W1" % i], p["c%d_b1" % i],
                          p["c%d_g1" % i], p["c%d_be1" % i])
        w2, b2 = _fold_bn(p["c%d_W2" % i], p["c%d_b2" % i],
                          p["c%d_g2" % i], p["c%d_be2" % i])
        h = _mlp2(h + aggr, w1, b1, w2, b2)
        hs.append(h)

    h_pool = jax.ops.segment_sum(h, batch, num_segments=G)
    h_pool_ = h_pool[batch]
    hcat = jnp.concatenate(hs + [h_pool_], axis=1)
    h = _mm(hcat, p["cl1_W"], p["cl1_b"])
    for j in range(2):
        h = _mm(h, p["cls%d_W" % j], p["cls%d_b" % j], kind="lrelu")
    return _mm(h, p["fin_W"], p["fin_b"], kind="sigmoid")
